# Initial kernel scaffold; baseline (speedup 1.0000x reference)
import jax
import jax.numpy as jnp
from jax.experimental import pallas as pl


def kernel(x, enc1_W, enc1_b, enc2_W, enc2_b, dec1_W, dec1_b, dec2_W, dec2_b, edge_index, mask_vector):
    raise NotImplementedError("write your pallas kernel here")


# speedup vs baseline: 9.5666x; 9.5666x over previous
import functools

import jax
import jax.numpy as jnp
from jax import lax
from jax.experimental import pallas as pl
from jax.experimental.pallas import tpu as pltpu
from jax.experimental.pallas import tpu_sc as plsc

NC, NS = 2, 16
NW = NC * NS
CH = 128


def _chunks(total, step):
    out = []
    while total > 0:
        out.append(min(step, total))
        total -= step
    return out


def _sc_mesh():
    return plsc.VectorSubcoreMesh(core_axis_name="c", subcore_axis_name="s",
                                  num_cores=NC, num_subcores=NS)


def _make_deg(N, EP):
    PC = EP // (CH * NW)
    NACC = ((N + 1 + 127) // 128) * 128
    ZR = NACC // NS
    F = 16

    @functools.partial(
        pl.kernel,
        out_type=jax.ShapeDtypeStruct((NC, NACC, F), jnp.float32),
        mesh=_sc_mesh(),
        scratch_types=[
            pltpu.VMEM((CH,), jnp.int32),
            pltpu.VMEM((CH, F), jnp.float32),
            pltpu.VMEM((CH, F), jnp.float32),
            pltpu.VMEM_SHARED((NACC, F), jnp.float32),
        ],
    )
    def deg(dstp, ones_hbm, zeros_hbm, out, didx, ones_v, rows, accum):
        c = lax.axis_index("c")
        s = lax.axis_index("s")
        w = s * NC + c
        pltpu.sync_copy(ones_hbm, ones_v)
        pltpu.sync_copy(zeros_hbm, rows)
        r0 = s * ZR
        off = 0
        for sz in _chunks(ZR, CH):
            pltpu.sync_copy(rows.at[pl.ds(0, sz)], accum.at[pl.ds(r0 + off, sz)])
            off += sz
        plsc.subcore_barrier()

        def step(i, carry):
            base = (w * PC + i) * CH
            pltpu.sync_copy(dstp.at[pl.ds(base, CH)], didx)
            pltpu.sync_copy(ones_v, accum.at[didx], add=True)
            return carry

        lax.fori_loop(0, PC, step, 0)
        plsc.subcore_barrier()
        off = 0
        for sz in _chunks(ZR, CH):
            pltpu.sync_copy(accum.at[pl.ds(r0 + off, sz)], rows.at[pl.ds(0, sz)])
            pltpu.sync_copy(rows.at[pl.ds(0, sz)], out.at[c, pl.ds(r0 + off, sz)])
            off += sz

    return deg


def _make_prop(N, F, EP):
    PC = EP // (CH * NW)
    NACC = ((N + 1 + 127) // 128) * 128
    ZR = NACC // NS

    @functools.partial(
        pl.kernel,
        out_type=jax.ShapeDtypeStruct((NC, NACC, F), jnp.float32),
        mesh=_sc_mesh(),
        scratch_types=[
            pltpu.VMEM((CH,), jnp.int32),
            pltpu.VMEM((CH,), jnp.int32),
            pltpu.VMEM((CH, F), jnp.float32),
            pltpu.VMEM_SHARED((NACC, F), jnp.float32),
            pltpu.SemaphoreType.DMA,
        ],
    )
    def prop(srcp, dstp, hs, zeros_hbm, out, sidx, didx, rows, accum, sem):
        c = lax.axis_index("c")
        s = lax.axis_index("s")
        w = s * NC + c
        pltpu.sync_copy(zeros_hbm, rows)
        r0 = s * ZR
        off = 0
        for sz in _chunks(ZR, CH):
            pltpu.sync_copy(rows.at[pl.ds(0, sz)], accum.at[pl.ds(r0 + off, sz)])
            off += sz
        plsc.subcore_barrier()

        def step(i, carry):
            base = (w * PC + i) * CH
            pltpu.sync_copy(srcp.at[pl.ds(base, CH)], sidx)
            pltpu.sync_copy(dstp.at[pl.ds(base, CH)], didx)
            pltpu.async_copy(hs.at[sidx], rows, sem).wait()
            pltpu.sync_copy(rows, accum.at[didx], add=True)
            return carry

        lax.fori_loop(0, PC, step, 0)
        plsc.subcore_barrier()
        off = 0
        for sz in _chunks(ZR, CH):
            pltpu.sync_copy(accum.at[pl.ds(r0 + off, sz)], rows.at[pl.ds(0, sz)])
            pltpu.sync_copy(rows.at[pl.ds(0, sz)], out.at[c, pl.ds(r0 + off, sz)])
            off += sz

    return prop


def _make_enc1(N, D, H, BN):
    grid = (N // BN,)

    def body(x_ref, m_ref, w_ref, p0_ref, p1_ref, hs1_ref, dinvb_ref):
        xb = x_ref[...] * m_ref[...]
        h1 = jnp.dot(xb, w_ref[...], preferred_element_type=jnp.float32)
        degv = p0_ref[:, 0:1] + p1_ref[:, 0:1] + 1.0
        dinv = 1.0 / jnp.sqrt(degv)
        hs1_ref[...] = h1 * dinv
        dinvb_ref[...] = jnp.broadcast_to(dinv, (BN, H))

    return pl.pallas_call(
        body,
        grid=grid,
        in_specs=[
            pl.BlockSpec((BN, D), lambda i: (i, 0)),
            pl.BlockSpec((BN, 1), lambda i: (i, 0)),
            pl.BlockSpec((D, H), lambda i: (0, 0)),
            pl.BlockSpec((BN, 16), lambda i: (i, 0)),
            pl.BlockSpec((BN, 16), lambda i: (i, 0)),
        ],
        out_specs=[
            pl.BlockSpec((BN, H), lambda i: (i, 0)),
            pl.BlockSpec((BN, H), lambda i: (i, 0)),
        ],
        out_shape=[
            jax.ShapeDtypeStruct((N, H), jnp.float32),
            jax.ShapeDtypeStruct((N, H), jnp.float32),
        ],
    )


def _make_enc2(N, H, L, BN):
    grid = (N // BN,)

    def body(pa_ref, pb_ref, hs1_ref, dv_ref, b1_ref, w_ref, hs2_ref):
        t = (pa_ref[...] + pb_ref[...] + hs1_ref[...]) * dv_ref[...] + b1_ref[...]
        h = jnp.maximum(t, 0.0)
        zp = jnp.dot(h, w_ref[...], preferred_element_type=jnp.float32)
        hs2_ref[...] = jnp.concatenate(
            [zp * dv_ref[:, 0:1], jnp.zeros((zp.shape[0], dv_ref.shape[1] - zp.shape[1]),
                                            jnp.float32)], axis=1)

    return pl.pallas_call(
        body,
        grid=grid,
        in_specs=[
            pl.BlockSpec((BN, H), lambda i: (i, 0)),
            pl.BlockSpec((BN, H), lambda i: (i, 0)),
            pl.BlockSpec((BN, H), lambda i: (i, 0)),
            pl.BlockSpec((BN, H), lambda i: (i, 0)),
            pl.BlockSpec((1, H), lambda i: (0, 0)),
            pl.BlockSpec((H, L), lambda i: (0, 0)),
        ],
        out_specs=pl.BlockSpec((BN, H), lambda i: (i, 0)),
        out_shape=jax.ShapeDtypeStruct((N, H), jnp.float32),
    )


def _make_dec(N, D, H, L, BN):
    grid = (N // BN,)

    def body(qa_ref, qb_ref, hs2_ref, dv_ref, b2_ref, w1_ref, b1_ref, w2_ref, b2d_ref,
             xrec_ref, z_ref):
        dinv = dv_ref[:, 0:1]
        z = (qa_ref[:, :L] + qb_ref[:, :L] + hs2_ref[:, :L]) * dinv + b2_ref[...]
        xr = jnp.maximum(jnp.dot(z, w1_ref[...], preferred_element_type=jnp.float32)
                         + b1_ref[...], 0.0)
        xrec_ref[...] = jnp.dot(xr, w2_ref[...], preferred_element_type=jnp.float32) + b2d_ref[...]
        z_ref[...] = z

    return pl.pallas_call(
        body,
        grid=grid,
        in_specs=[
            pl.BlockSpec((BN, H), lambda i: (i, 0)),
            pl.BlockSpec((BN, H), lambda i: (i, 0)),
            pl.BlockSpec((BN, H), lambda i: (i, 0)),
            pl.BlockSpec((BN, H), lambda i: (i, 0)),
            pl.BlockSpec((1, L), lambda i: (0, 0)),
            pl.BlockSpec((L, H), lambda i: (0, 0)),
            pl.BlockSpec((1, H), lambda i: (0, 0)),
            pl.BlockSpec((H, D), lambda i: (0, 0)),
            pl.BlockSpec((1, D), lambda i: (0, 0)),
        ],
        out_specs=[
            pl.BlockSpec((BN, D), lambda i: (i, 0)),
            pl.BlockSpec((BN, L), lambda i: (i, 0)),
        ],
        out_shape=[
            jax.ShapeDtypeStruct((N, D), jnp.float32),
            jax.ShapeDtypeStruct((N, L), jnp.float32),
        ],
    )


def kernel(x, enc1_W, enc1_b, enc2_W, enc2_b, dec1_W, dec1_b, dec2_W, dec2_b,
           edge_index, mask_vector):
    N, D = x.shape
    H = enc1_W.shape[1]
    L = enc2_W.shape[1]
    E = edge_index.shape[1]
    BN = 1000

    EP = -(-E // (CH * NW)) * (CH * NW)
    pad = EP - E
    src_p = jnp.concatenate([edge_index[0], jnp.zeros((pad,), jnp.int32)])
    dst_p = jnp.concatenate([edge_index[1], jnp.full((pad,), N, jnp.int32)])

    ones16 = jnp.ones((CH, 16), jnp.float32)
    zeros16 = jnp.zeros((CH, 16), jnp.float32)
    zerosH = jnp.zeros((CH, H), jnp.float32)

    degp = _make_deg(N, EP)(dst_p, ones16, zeros16)
    hs1, dinvb = _make_enc1(N, D, H, BN)(x, mask_vector, enc1_W, degp[0], degp[1])
    p = _make_prop(N, H, EP)(src_p, dst_p, hs1, zerosH)
    hs2 = _make_enc2(N, H, L, BN)(p[0], p[1], hs1, dinvb,
                                  enc1_b.reshape(1, H), enc2_W)
    q = _make_prop(N, H, EP)(src_p, dst_p, hs2, zerosH)
    x_recon, z = _make_dec(N, D, H, L, BN)(q[0], q[1], hs2, dinvb,
                                           enc2_b.reshape(1, L), dec1_W,
                                           dec1_b.reshape(1, H), dec2_W,
                                           dec2_b.reshape(1, D))
    return (x_recon, z)
